# Initial kernel scaffold; baseline (speedup 1.0000x reference)
import jax
import jax.numpy as jnp
from jax.experimental import pallas as pl


def kernel(x, MPS):
    raise NotImplementedError("write your pallas kernel here")


# speedup vs baseline: 3.5401x; 3.5401x over previous
import functools

import jax
import jax.numpy as jnp
from jax.experimental import pallas as pl
from jax.experimental.pallas import tpu as pltpu

N_SITES = 784
D = 128
B = 256
S = 16
NBLK = N_SITES // S


def _mps_body(mcat_ref, xft_ref, out_ref, alt_ref, gl_ref):
    p = pl.program_id(0)
    j = pl.program_id(1)

    @pl.when(j == 0)
    def _init():
        row = jax.lax.broadcasted_iota(jnp.int32, (D, B), 0)
        alt_ref[...] = jnp.where(row == 0, 1.0, 0.0)
        rowg = jax.lax.broadcasted_iota(jnp.int32, (D, D), 0)
        colg = jax.lax.broadcasted_iota(jnp.int32, (D, D), 1)
        gl_ref[...] = jnp.where((rowg == 0) & (colg == 0), 1.0, 0.0)

    @pl.when(p == 0)
    def _batch_scan():
        xblk = xft_ref[...]
        e0b = jnp.cos(0.5 * jnp.pi * xblk)
        e1b = jnp.sin(0.5 * jnp.pi * xblk)

        alt = alt_ref[...]
        for s in range(S):
            m = mcat_ref[s]
            yv = jax.lax.dot_general(
                m, alt, (((0,), (0,)), ((), ())),
                preferred_element_type=jnp.float32)
            alt = yv[:D] * e0b[s:s + 1] + yv[D:] * e1b[s:s + 1]
        alt_ref[...] = alt

        @pl.when(j == NBLK - 1)
        def _():
            out_ref[0] = alt

    @pl.when(p == 1)
    def _norm_scan():
        gl = gl_ref[...]
        for s in range(S):
            m = mcat_ref[s]
            yv = jax.lax.dot_general(
                m, gl, (((0,), (0,)), ((), ())),
                preferred_element_type=jnp.float32)
            r0 = jnp.dot(yv[:D], m[:, :D], preferred_element_type=jnp.float32)
            r1 = jnp.dot(yv[D:], m[:, D:], preferred_element_type=jnp.float32)
            gl = r0 + r1
        gl_ref[...] = gl

        @pl.when(j == NBLK - 1)
        def _():
            out_ref[0, :, :D] = gl


@functools.partial(jax.jit, static_argnames=("interpret",))
def kernel(x, MPS, interpret=False):
    xft = x.reshape(B, -1).T
    mcat = jnp.concatenate([MPS[..., 0], MPS[..., 1]], axis=-1)

    buf = pl.pallas_call(
        _mps_body,
        grid=(2, NBLK),
        in_specs=[
            pl.BlockSpec((S, D, 2 * D), lambda p, j: (j, 0, 0)),
            pl.BlockSpec((S, B), lambda p, j: (j, 0)),
        ],
        out_specs=pl.BlockSpec((1, D, B), lambda p, j: (p, 0, 0)),
        out_shape=jax.ShapeDtypeStruct((2, D, B), jnp.float32),
        scratch_shapes=[
            pltpu.VMEM((D, B), jnp.float32),
            pltpu.VMEM((D, D), jnp.float32),
        ],
        compiler_params=pltpu.CompilerParams(
            dimension_semantics=("parallel", "arbitrary"),
        ),
        interpret=interpret,
    )(mcat, xft)

    amp = buf[0, 0, :]
    norm_sq = buf[1, 0, 0]
    return amp * amp / norm_sq
